# speedup vs baseline: 2.9552x; 2.9552x over previous
import jax
import jax.numpy as jnp
from jax.experimental import pallas as pl

_BATCH = 16384
_ROWS = 128
_COLS = 128


def _item2vec_kernel(x_ref, y_ref, tab_ref, out_ref):
    t = tab_ref[...]
    n = t.shape[0]
    xc = jnp.clip(x_ref[...], 0, n - 1)
    yc = jnp.clip(y_ref[...], 0, n - 1)
    dots = jnp.zeros(out_ref.shape, jnp.float32)
    for r in range(n):
        for q in range(n):
            g = jnp.sum(t[r, :] * t[q, :])
            m = jnp.logical_and(xc == r, yc == q)
            dots = dots + jnp.where(m, g, 0.0)
    out_ref[...] = jax.nn.sigmoid(dots)


def kernel(batch_data, table):
    x = batch_data[0].reshape(_ROWS, _COLS)
    y = batch_data[1].reshape(_ROWS, _COLS)
    out = pl.pallas_call(
        _item2vec_kernel,
        out_shape=jax.ShapeDtypeStruct((_ROWS, _COLS), jnp.float32),
    )(x, y, table)
    return out.reshape(_BATCH)
